# Initial kernel scaffold; baseline (speedup 1.0000x reference)
import jax
import jax.numpy as jnp
from jax.experimental import pallas as pl


def kernel(x, edge_index, W1, b1, W2, b2):
    raise NotImplementedError("write your pallas kernel here")


# speedup vs baseline: 9.4200x; 9.4200x over previous
import functools

import jax
import jax.numpy as jnp
from jax import lax
from jax.experimental import pallas as pl
from jax.experimental.pallas import tpu as pltpu
from jax.experimental.pallas import tpu_sc as plsc

NC, NS, L = 2, 16, 16
NW = NC * NS


@functools.lru_cache(maxsize=None)
def _prop(npad, d, e):
  epw = e // NW
  batch = 80
  nb = epw // batch
  rpw = npad // NS
  mesh = plsc.VectorSubcoreMesh(
      core_axis_name="c", subcore_axis_name="s", num_cores=NC, num_subcores=NS)

  def body(h_hbm, src_hbm, dst_hbm, zeros_hbm, part_hbm,
           sidx, didx, rows, acc_sh, sem):
    c = lax.axis_index("c")
    s = lax.axis_index("s")
    w = s * NC + c
    pltpu.sync_copy(zeros_hbm.at[pl.ds(s * rpw, rpw)],
                    acc_sh.at[pl.ds(s * rpw, rpw)])
    plsc.subcore_barrier()
    base = w * epw

    def step(b, carry):
      off = base + b * batch
      pltpu.sync_copy(src_hbm.at[pl.ds(off, batch)], sidx)
      pltpu.sync_copy(dst_hbm.at[pl.ds(off, batch)], didx)
      pltpu.async_copy(h_hbm.at[sidx], rows, sem).wait()
      pltpu.sync_copy(rows, acc_sh.at[didx], add=True)
      return carry

    lax.fori_loop(0, nb, step, 0)
    plsc.subcore_barrier()
    pltpu.sync_copy(acc_sh.at[pl.ds(s * rpw, rpw)],
                    part_hbm.at[c, pl.ds(s * rpw, rpw)])

  return pl.kernel(
      body,
      out_type=jax.ShapeDtypeStruct((NC, npad, d), jnp.float32),
      mesh=mesh,
      compiler_params=pltpu.CompilerParams(use_tc_tiling_on_sc=(d == 128)),
      scratch_types=[
          pltpu.VMEM((batch,), jnp.int32),
          pltpu.VMEM((batch,), jnp.int32),
          pltpu.VMEM((batch, d), jnp.float32),
          pltpu.VMEM_SHARED((npad, d), jnp.float32),
          pltpu.SemaphoreType.DMA,
      ])


def _dinv(degp_ref):
  deg = degp_ref[0][:, 0:1] + degp_ref[1][:, 0:1]
  return jnp.where(deg > 0, lax.rsqrt(jnp.maximum(deg, 1.0)), 0.0)


def _tc1(x, w1, b1, degp):
  n = x.shape[0]
  r = 1000

  def body(x_ref, w_ref, b_ref, degp_ref, out_ref):
    dinv = _dinv(degp_ref)
    h = jnp.dot(x_ref[...], w_ref[...], preferred_element_type=jnp.float32)
    out_ref[...] = (h + b_ref[...]) * dinv

  return pl.pallas_call(
      body,
      grid=(n // r,),
      in_specs=[
          pl.BlockSpec((r, 128), lambda i: (i, 0)),
          pl.BlockSpec((128, 128), lambda i: (0, 0)),
          pl.BlockSpec((1, 128), lambda i: (0, 0)),
          pl.BlockSpec((NC, r, 16), lambda i: (0, i, 0)),
      ],
      out_specs=pl.BlockSpec((r, 128), lambda i: (i, 0)),
      out_shape=jax.ShapeDtypeStruct((n, 128), jnp.float32),
  )(x, w1, b1.reshape(1, 128), degp)


def _tc2(n, p, degp, w2, b2):
  r = 1000

  def body(p_ref, degp_ref, w_ref, b_ref, out_ref):
    dinv = _dinv(degp_ref)
    t = jnp.maximum((p_ref[0] + p_ref[1]) * dinv, 0.0)
    h = jnp.dot(t, w_ref[...], preferred_element_type=jnp.float32)
    out_ref[...] = (h + b_ref[...]) * dinv

  return pl.pallas_call(
      body,
      grid=(n // r,),
      in_specs=[
          pl.BlockSpec((NC, r, 128), lambda i: (0, i, 0)),
          pl.BlockSpec((NC, r, 16), lambda i: (0, i, 0)),
          pl.BlockSpec((128, 128), lambda i: (0, 0)),
          pl.BlockSpec((1, 128), lambda i: (0, 0)),
      ],
      out_specs=pl.BlockSpec((r, 128), lambda i: (i, 0)),
      out_shape=jax.ShapeDtypeStruct((n, 128), jnp.float32),
  )(p, degp, w2, b2.reshape(1, 128))


def _tc3(n, q, degp):
  r = 1000

  def body(q_ref, degp_ref, out_ref):
    dinv = _dinv(degp_ref)
    out_ref[...] = (q_ref[0] + q_ref[1]) * dinv

  return pl.pallas_call(
      body,
      grid=(n // r,),
      in_specs=[
          pl.BlockSpec((NC, r, 128), lambda i: (0, i, 0)),
          pl.BlockSpec((NC, r, 16), lambda i: (0, i, 0)),
      ],
      out_specs=pl.BlockSpec((r, 128), lambda i: (i, 0)),
      out_shape=jax.ShapeDtypeStruct((n, 128), jnp.float32),
  )(q, degp)


def kernel(x, edge_index, W1, b1, W2, b2):
  n = x.shape[0]
  src = edge_index[0].astype(jnp.int32)
  dst = edge_index[1].astype(jnp.int32)
  e = src.shape[0]
  npad = ((n + 8 * NS - 1) // (8 * NS)) * (8 * NS)

  ones16 = jnp.ones((n, 16), jnp.float32)
  z16 = jnp.zeros((npad, 16), jnp.float32)
  z128 = jnp.zeros((npad, 128), jnp.float32)

  degp = _prop(npad, 16, e)(ones16, src, dst, z16)
  h1s = _tc1(x, W1, b1, degp)
  p = _prop(npad, 128, e)(h1s, src, dst, z128)
  h2s = _tc2(n, p, degp, W2, b2)
  q = _prop(npad, 128, e)(h2s, src, dst, z128)
  return _tc3(n, q, degp)
